# Initial kernel scaffold; baseline (speedup 1.0000x reference)
import jax
import jax.numpy as jnp
from jax.experimental import pallas as pl


def kernel(x, edge_index, edge_type, W_msg, b_msg, W_ih, W_hh, b_ih, b_hh, W_cls, b_cls):
    raise NotImplementedError("write your pallas kernel here")


# speedup vs baseline: 1.9020x; 1.9020x over previous
import functools

import jax
import jax.numpy as jnp
from jax import lax
from jax.experimental import pallas as pl
from jax.experimental.pallas import tpu as pltpu
from jax.experimental.pallas import tpu_sc as plsc

N_STEPS = 6

NC = 2
NS = 16
CHUNK = 128


def _hall_body(h_ref, w_ref, b_ref, out_ref):
    w = w_ref[0]
    out_ref[0] = lax.dot_general(
        h_ref[...], w, (((1,), (1,)), ((), ())),
        preferred_element_type=jnp.float32) + b_ref[0]


def _hall_call(h, W_msg, b_msg, *, n_blk):
    N, D = h.shape
    K = W_msg.shape[0]
    nb = N // n_blk
    return pl.pallas_call(
        _hall_body,
        grid=(nb, K),
        in_specs=[
            pl.BlockSpec((n_blk, D), lambda i, k: (i, 0)),
            pl.BlockSpec((1, D, D), lambda i, k: (k, 0, 0)),
            pl.BlockSpec((1, 1, D), lambda i, k: (k, 0, 0)),
        ],
        out_specs=pl.BlockSpec((1, n_blk, D), lambda i, k: (k, i, 0)),
        out_shape=jax.ShapeDtypeStruct((K, N, D), jnp.float32),
    )(h, W_msg, b_msg[:, None, :])


def _sc_body(n_pad, cpw, hall_ref, gidx_ref, dst_ref, zeros_ref,
             out_ref, idx_v, dst_v, buf_v, acc_sh):
    c = lax.axis_index("c")
    s = lax.axis_index("s")
    wid = s * NC + c

    rows_per_tile = n_pad // NS
    pltpu.sync_copy(zeros_ref.at[pl.ds(s * rows_per_tile, rows_per_tile)],
                    acc_sh.at[pl.ds(s * rows_per_tile, rows_per_tile)])
    plsc.subcore_barrier()

    def body(j, carry):
        base = (wid * cpw + j) * CHUNK
        pltpu.sync_copy(gidx_ref.at[pl.ds(base, CHUNK)], idx_v)
        pltpu.sync_copy(dst_ref.at[pl.ds(base, CHUNK)], dst_v)
        pltpu.sync_copy(hall_ref.at[idx_v], buf_v)
        pltpu.sync_copy(buf_v, acc_sh.at[dst_v], add=True)
        return carry

    lax.fori_loop(0, cpw, body, 0)
    plsc.subcore_barrier()

    pltpu.sync_copy(acc_sh.at[pl.ds(s * rows_per_tile, rows_per_tile)],
                    out_ref.at[c, pl.ds(s * rows_per_tile, rows_per_tile)])


def _sc_call(hall_flat, gidx, dst, zeros_np, *, n, d, n_pad, cpw):
    mesh = plsc.VectorSubcoreMesh(core_axis_name="c", subcore_axis_name="s")
    body = functools.partial(_sc_body, n_pad, cpw)
    return pl.kernel(
        body,
        out_type=jax.ShapeDtypeStruct((NC, n_pad, d), jnp.float32),
        mesh=mesh,
        scratch_types=[
            pltpu.VMEM((CHUNK,), jnp.int32),
            pltpu.VMEM((CHUNK,), jnp.int32),
            pltpu.VMEM((CHUNK, d), jnp.float32),
            pltpu.VMEM_SHARED((n_pad, d), jnp.float32),
        ],
    )(hall_flat, gidx, dst, zeros_np)


def _gru_body(a01_ref, h_ref, wih_ref, whh_ref, bih_ref, bhh_ref,
              wcls_ref, bcls_ref, hnew_ref, logit_ref):
    i = pl.program_id(0)
    nb = pl.num_programs(0)
    a = a01_ref[0] + a01_ref[1]
    h = h_ref[...]
    gi = lax.dot_general(a, wih_ref[...], (((1,), (1,)), ((), ())),
                         preferred_element_type=jnp.float32) + bih_ref[...]
    gh = lax.dot_general(h, whh_ref[...], (((1,), (1,)), ((), ())),
                         preferred_element_type=jnp.float32) + bhh_ref[...]
    D = h.shape[1]
    r = jax.nn.sigmoid(gi[:, :D] + gh[:, :D])
    z = jax.nn.sigmoid(gi[:, D:2 * D] + gh[:, D:2 * D])
    n = jnp.tanh(gi[:, 2 * D:] + r * gh[:, 2 * D:])
    hn = (1.0 - z) * n + z * h
    hnew_ref[...] = hn

    @pl.when(i == 0)
    def _():
        logit_ref[...] = jnp.zeros_like(logit_ref)

    logit_ref[...] += jnp.sum(hn, axis=0, keepdims=True)

    @pl.when(i == nb - 1)
    def _():
        hg = logit_ref[...]
        logit_ref[...] = lax.dot_general(
            hg, wcls_ref[...], (((1,), (1,)), ((), ())),
            preferred_element_type=jnp.float32) + bcls_ref[...]


def _gru_call(a01, h, W_ih, W_hh, b_ih, b_hh, wcls_pad, bcls_pad, *, n_blk):
    N, D = h.shape
    nb = N // n_blk
    return pl.pallas_call(
        _gru_body,
        grid=(nb,),
        in_specs=[
            pl.BlockSpec((2, n_blk, D), lambda i: (0, i, 0)),
            pl.BlockSpec((n_blk, D), lambda i: (i, 0)),
            pl.BlockSpec((3 * D, D), lambda i: (0, 0)),
            pl.BlockSpec((3 * D, D), lambda i: (0, 0)),
            pl.BlockSpec((1, 3 * D), lambda i: (0, 0)),
            pl.BlockSpec((1, 3 * D), lambda i: (0, 0)),
            pl.BlockSpec((D, D), lambda i: (0, 0)),
            pl.BlockSpec((1, D), lambda i: (0, 0)),
        ],
        out_specs=[
            pl.BlockSpec((n_blk, D), lambda i: (i, 0)),
            pl.BlockSpec((1, D), lambda i: (0, 0)),
        ],
        out_shape=[
            jax.ShapeDtypeStruct((N, D), jnp.float32),
            jax.ShapeDtypeStruct((1, D), jnp.float32),
        ],
    )(a01, h, W_ih, W_hh, b_ih, b_hh, wcls_pad, bcls_pad)


def kernel(x, edge_index, edge_type, W_msg, b_msg, W_ih, W_hh, b_ih, b_hh,
           W_cls, b_cls):
    N, D = x.shape
    K = W_msg.shape[0]
    E = edge_index.shape[1]
    n_cls = W_cls.shape[0]

    src = edge_index[0]
    dst = edge_index[1]
    gidx = edge_type * N + src

    n_workers = NC * NS
    e_pad = ((E + n_workers * CHUNK - 1) // (n_workers * CHUNK)) * (n_workers * CHUNK)
    cpw = e_pad // (n_workers * CHUNK)
    gidx = jnp.concatenate([gidx, jnp.zeros((e_pad - E,), jnp.int32)])
    dst = jnp.concatenate([dst, jnp.full((e_pad - E,), N, jnp.int32)])

    n_pad = ((N + 1 + NS * 8 - 1) // (NS * 8)) * (NS * 8)
    zeros_np = jnp.zeros((n_pad, D), jnp.float32)

    bih2 = b_ih.reshape(1, 3 * D)
    bhh2 = b_hh.reshape(1, 3 * D)
    wcls_pad = jnp.zeros((D, D), jnp.float32).at[:n_cls].set(W_cls)
    bcls_pad = jnp.zeros((1, D), jnp.float32).at[0, :n_cls].set(b_cls)

    n_blk = 1000
    h = x
    logits = None
    for _ in range(N_STEPS):
        hall = _hall_call(h, W_msg, b_msg, n_blk=n_blk)
        a01 = _sc_call(hall.reshape(K * N, D), gidx, dst, zeros_np,
                       n=N, d=D, n_pad=n_pad, cpw=cpw)
        h, logits = _gru_call(a01, h, W_ih, W_hh, bih2, bhh2,
                              wcls_pad, bcls_pad, n_blk=n_blk)
    return logits[:, :n_cls]
